# Initial kernel scaffold; baseline (speedup 1.0000x reference)
import jax
import jax.numpy as jnp
from jax.experimental import pallas as pl


def kernel(x, down_neigh_indices, down_indices):
    raise NotImplementedError("write your pallas kernel here")


# speedup vs baseline: 17.9814x; 17.9814x over previous
import functools

import jax
import jax.numpy as jnp
from jax import lax
from jax.experimental import pallas as pl
from jax.experimental.pallas import tpu as pltpu
from jax.experimental.pallas import tpu_sc as plsc

NC = 2
NS = 16
NW = NC * NS
L = 16
NEIGH = 7
G = 56


@functools.lru_cache(maxsize=None)
def _gather_mean(n_out_pad: int, n_chunks: int, d: int):
  mesh = plsc.VectorSubcoreMesh(core_axis_name="c", subcore_axis_name="s")

  @functools.partial(
      pl.kernel,
      out_type=jax.ShapeDtypeStruct((n_out_pad, d), jnp.float32),
      mesh=mesh,
      scratch_types=[
          pltpu.VMEM((NEIGH, G), jnp.int32),
          pltpu.VMEM((NEIGH, G, d), jnp.float32),
          pltpu.VMEM((G, d), jnp.float32),
          pltpu.SemaphoreType.DMA,
          pltpu.SemaphoreType.DMA,
      ],
  )
  def kern(xt_hbm, dnit_hbm, out_hbm, idx_v, rows_v, out_v, gsem, osem):
    wid = lax.axis_index("s") * NC + lax.axis_index("c")

    def chunk_body(ci, carry):
      base = (wid * n_chunks + ci) * G
      for k in range(NEIGH):
        pltpu.sync_copy(dnit_hbm.at[pl.ds(k * n_out_pad + base, G)], idx_v.at[k])
      copies = [
          pltpu.async_copy(xt_hbm.at[idx_v.at[k]], rows_v.at[k], gsem)
          for k in range(NEIGH)
      ]
      for c in copies:
        c.wait()

      def row_body(r, _):
        def col_body(j, __):
          sl = pl.ds(j * L, L)
          acc = rows_v[0, r, sl]
          for k in range(1, NEIGH):
            acc = acc + rows_v[k, r, sl]
          out_v[r, sl] = acc * (1.0 / NEIGH)
          return __
        return lax.fori_loop(0, d // L, col_body, _, unroll=4)

      lax.fori_loop(0, G, row_body, 0)
      pltpu.async_copy(out_v, out_hbm.at[pl.ds(base, G)], osem).wait()
      return carry

    lax.fori_loop(0, n_chunks, chunk_body, 0)

  return kern


def kernel(x, down_neigh_indices, down_indices):
  b, c, n_in = x.shape
  d = b * c
  n_out = down_indices.shape[0]

  dni = jnp.take(down_neigh_indices, down_indices, axis=0)
  per_w = -(-n_out // NW)
  n_chunks = -(-per_w // G)
  n_out_pad = NW * n_chunks * G
  dni_t = jnp.pad(dni.T.astype(jnp.int32),
                  ((0, 0), (0, n_out_pad - n_out))).reshape(-1)

  xt = jnp.transpose(x.reshape(d, n_in))

  out_t = _gather_mean(n_out_pad, n_chunks, d)(xt, dni_t)
  return jnp.transpose(out_t[:n_out]).reshape(b, c, n_out)
